# baseline (speedup 1.0000x reference)
import functools

import jax
import jax.numpy as jnp
from jax import lax
from jax.experimental import pallas as pl
from jax.experimental.pallas import tpu as pltpu
from jax.experimental.pallas import tpu_sc as plsc

B = 16
S = 512
NLM = 5000
NLM_PAD = 5120
NPAIRS = B * NLM
NTILES = 32
LANES = 16
PAIRS = NPAIRS // NTILES
NBLK = 21
WWORDS = NBLK * 256
NV = NBLK * 8
WPAIR = NBLK * 128
TOTAL = NPAIRS * 2
PLANE = S * S


def _sc_body(flow_hbm, lmf_hbm, lms_hbm, gate_hbm, out_hbm,
             lmf_v, lms_v, gate_v, idx_v, pts_v, row_v, sem):
    cid = lax.axis_index("c")
    sid = lax.axis_index("s")
    wid = cid * 16 + sid
    batch = wid // 2
    half = wid % 2
    b0 = half * (NLM_PAD // 128 - NBLK)
    j_lo = half * PAIRS
    base = batch * (2 * NLM_PAD) + b0 * 256
    plane0 = batch * (2 * PLANE)

    pltpu.sync_copy(lmf_hbm.at[pl.ds(base, WWORDS)], lmf_v)
    pltpu.sync_copy(lms_hbm.at[pl.ds(base, WWORDS)], lms_v)
    pltpu.sync_copy(gate_hbm.at[pl.ds(base, WWORDS)], gate_v)

    lanes = lax.iota(jnp.int32, 16)

    def idx_body(v, _):
        off = ((v >> 3) << 8) + ((v & 7) << 4)
        j = (b0 << 7) + ((v >> 3) << 7) + ((v & 7) << 4) + lanes
        valid = (j >= j_lo) & (j < j_lo + PAIRS)
        x = lmf_v[pl.ds(off, LANES)]
        y = lmf_v[pl.ds(off + 128, LANES)]
        within = (((y >> 3) * 4 + (x >> 7)) << 10) + ((y & 7) << 7) + (x & 127)
        idx0 = jnp.where(valid, plane0 + within, 0)
        idx_v[pl.ds(v * LANES, LANES)] = idx0
        idx_v[pl.ds(NV * LANES + v * LANES, LANES)] = idx0 + PLANE
        return 0

    lax.fori_loop(0, NV, idx_body, 0)

    pltpu.async_copy(flow_hbm.at[idx_v], pts_v, sem).wait()

    inv_half_s = jnp.float32(2.0 / S)
    scale = jnp.float32(1.0 / TOTAL)

    def acc_body(v, acc):
        off = ((v >> 3) << 8) + ((v & 7) << 4)
        j = (b0 << 7) + ((v >> 3) << 7) + ((v & 7) << 4) + lanes
        valid = (j >= j_lo) & (j < j_lo + PAIRS)
        g0 = gate_v[pl.ds(off, LANES)]
        g1 = gate_v[pl.ds(off + 128, LANES)]
        s0 = lms_v[pl.ds(off, LANES)]
        s1 = lms_v[pl.ds(off + 128, LANES)]
        gt0 = s0.astype(jnp.float32) * inv_half_s - 1.0
        gt1 = s1.astype(jnp.float32) * inv_half_s - 1.0
        pt0 = pts_v[pl.ds(v * LANES, LANES)]
        pt1 = pts_v[pl.ds(NV * LANES + v * LANES, LANES)]
        d0 = pt0 * g0 - gt0 * g0
        d1 = pt1 * g1 - gt1 * g1
        sq = d0 * d0 + d1 * d1
        return acc + jnp.where(valid, sq, jnp.float32(0.0))

    acc = lax.fori_loop(0, NV, acc_body, jnp.zeros((16,), jnp.float32))
    row_v[...] = acc * scale
    pltpu.sync_copy(row_v, out_hbm.at[wid])


@jax.jit
def _landmark_loss(flow_flat, lmf_p, lms_p, gate_p):
    mesh = plsc.VectorSubcoreMesh(core_axis_name="c", subcore_axis_name="s")
    run = functools.partial(
        pl.kernel,
        out_type=jax.ShapeDtypeStruct((NTILES, 16), jnp.float32),
        mesh=mesh,
        scratch_types=[
            pltpu.VMEM((WWORDS,), jnp.int32),
            pltpu.VMEM((WWORDS,), jnp.int32),
            pltpu.VMEM((WWORDS,), jnp.float32),
            pltpu.VMEM((2 * NV * LANES,), jnp.int32),
            pltpu.VMEM((2 * NV * LANES,), jnp.float32),
            pltpu.VMEM((16,), jnp.float32),
            pltpu.SemaphoreType.DMA,
        ],
        compiler_params=pltpu.CompilerParams(needs_layout_passes=False),
    )(_sc_body)
    partials = run(flow_flat, lmf_p, lms_p, gate_p)
    return jnp.sum(partials)


def _tile_view(a):
    p = jnp.pad(a, ((0, 0), (0, NLM_PAD - NLM), (0, 0)))
    return p.reshape(B, NLM_PAD // 128, 128, 2).transpose(0, 1, 3, 2).reshape(-1)


def kernel(flow, lm_S, lm_F, gate):
    flow_t = (
        flow.reshape(B, 2, S // 8, 8, S // 128, 128)
        .transpose(0, 1, 2, 4, 3, 5)
        .reshape(-1)
    )
    return _landmark_loss(flow_t, _tile_view(lm_F), _tile_view(lm_S),
                          _tile_view(gate))


# speedup vs baseline: 1.7848x; 1.7848x over previous
import functools

import jax
import jax.numpy as jnp
from jax import lax
from jax.experimental import pallas as pl
from jax.experimental.pallas import tpu as pltpu
from jax.experimental.pallas import tpu_sc as plsc

B = 16
S = 512
NLM = 5000
NPAIRS = B * NLM
NTILES = 32
LANES = 16
PAIRS = NPAIRS // NTILES
WIN = 2504
PAD = 2512
NVEC = PAD // LANES
TOTAL = NPAIRS * 2
PLANE = S * S


def _sc_body(flow_hbm, x_hbm, y_hbm, sx_hbm, sy_hbm, g0_hbm, g1_hbm, out_hbm,
             x_v, y_v, sx_v, sy_v, g0_v, g1_v, idx_v, pts_v, row_v, sem):
    cid = lax.axis_index("c")
    sid = lax.axis_index("s")
    wid = cid * 16 + sid
    batch = wid // 2
    half = wid % 2
    w0 = half * (NLM - WIN)
    lo = half * 4
    plane0 = batch * (2 * PLANE)

    for src, buf in zip((x_hbm, y_hbm, sx_hbm, sy_hbm, g0_hbm, g1_hbm),
                        (x_v, y_v, sx_v, sy_v, g0_v, g1_v)):
        pltpu.sync_copy(src.at[pl.ds(batch * NLM + w0, WIN)],
                        buf.at[pl.ds(0, WIN)])

    lanes = lax.iota(jnp.int32, 16)

    def idx_at(v, masked):
        off = v * LANES
        x = x_v[pl.ds(off, LANES)]
        y = y_v[pl.ds(off, LANES)]
        within = (((y >> 3) * 4 + (x >> 7)) << 10) + ((y & 7) << 7) + (x & 127)
        idx0 = plane0 + within
        if masked:
            idx0 = jnp.where(off + lanes < WIN, idx0, 0)
        idx_v[pl.ds(off, LANES)] = idx0
        idx_v[pl.ds(PAD + off, LANES)] = idx0 + PLANE

    def idx_body(v, _):
        idx_at(v, False)
        return 0

    idx_at(0, False)
    lax.fori_loop(1, NVEC - 1, idx_body, 0)
    idx_at(NVEC - 1, True)

    pltpu.async_copy(flow_hbm.at[idx_v], pts_v, sem).wait()

    inv_half_s = jnp.float32(2.0 / S)
    scale = jnp.float32(1.0 / TOTAL)

    def sq_at(v, masked):
        off = v * LANES
        g0 = plsc.bitcast(g0_v[pl.ds(off, LANES)], jnp.float32)
        g1 = plsc.bitcast(g1_v[pl.ds(off, LANES)], jnp.float32)
        s0 = sx_v[pl.ds(off, LANES)]
        s1 = sy_v[pl.ds(off, LANES)]
        gt0 = s0.astype(jnp.float32) * inv_half_s - 1.0
        gt1 = s1.astype(jnp.float32) * inv_half_s - 1.0
        pt0 = pts_v[pl.ds(off, LANES)]
        pt1 = pts_v[pl.ds(PAD + off, LANES)]
        d0 = pt0 * g0 - gt0 * g0
        d1 = pt1 * g1 - gt1 * g1
        sq = d0 * d0 + d1 * d1
        if masked:
            p = off + lanes
            sq = jnp.where((p >= lo) & (p < lo + PAIRS), sq, jnp.float32(0.0))
        return sq

    def acc_body(v, acc):
        return acc + sq_at(v, False)

    acc = sq_at(0, True)
    acc = lax.fori_loop(1, NVEC - 1, acc_body, acc)
    acc = acc + sq_at(NVEC - 1, True)

    row_v[...] = acc * scale
    pltpu.sync_copy(row_v, out_hbm.at[wid])


@jax.jit
def _landmark_loss(flow_flat, x_f, y_f, sx_f, sy_f, g0_f, g1_f):
    mesh = plsc.VectorSubcoreMesh(core_axis_name="c", subcore_axis_name="s")
    run = functools.partial(
        pl.kernel,
        out_type=jax.ShapeDtypeStruct((NTILES, 16), jnp.float32),
        mesh=mesh,
        scratch_types=[
            pltpu.VMEM((PAD,), jnp.int32),
            pltpu.VMEM((PAD,), jnp.int32),
            pltpu.VMEM((PAD,), jnp.int32),
            pltpu.VMEM((PAD,), jnp.int32),
            pltpu.VMEM((PAD,), jnp.int32),
            pltpu.VMEM((PAD,), jnp.int32),
            pltpu.VMEM((2 * PAD,), jnp.int32),
            pltpu.VMEM((2 * PAD,), jnp.float32),
            pltpu.VMEM((16,), jnp.float32),
            pltpu.SemaphoreType.DMA,
        ],
        compiler_params=pltpu.CompilerParams(needs_layout_passes=False),
    )(_sc_body)
    partials = run(flow_flat, x_f, y_f, sx_f, sy_f, g0_f, g1_f)
    return jnp.sum(partials)


def kernel(flow, lm_S, lm_F, gate):
    flow_t = (
        flow.reshape(B, 2, S // 8, 8, S // 128, 128)
        .transpose(0, 1, 2, 4, 3, 5)
        .reshape(-1)
    )
    gate_i = jax.lax.bitcast_convert_type(gate, jnp.int32)
    return _landmark_loss(
        flow_t,
        lm_F[:, :, 0].reshape(-1),
        lm_F[:, :, 1].reshape(-1),
        lm_S[:, :, 0].reshape(-1),
        lm_S[:, :, 1].reshape(-1),
        gate_i[:, :, 0].reshape(-1),
        gate_i[:, :, 1].reshape(-1),
    )
